# Initial kernel scaffold; baseline (speedup 1.0000x reference)
import jax
import jax.numpy as jnp
from jax.experimental import pallas as pl


def kernel(x, index):
    raise NotImplementedError("write your pallas kernel here")


# speedup vs baseline: 14.7462x; 14.7462x over previous
import jax
import jax.numpy as jnp
from jax import lax
from jax.experimental import pallas as pl
from jax.experimental.pallas import tpu as pltpu
from jax.experimental.pallas import tpu_sc as plsc

B = 16384
D = 4096
K = 128
NC, NS = 2, 16
NW = NC * NS
ROWS_PER_W = B // NW
R = 8
CHUNKS = ROWS_PER_W // R
VPR = K // 16


def _body(x_hbm, idx_hbm, out_hbm,
          data0, data1, idxb0, idxb1,
          sd0, sd1, si0, si1, so0, so1):
    wid = lax.axis_index("s") * NC + lax.axis_index("c")
    row0 = wid * ROWS_PER_W

    data = (data0, data1)
    idxb = (idxb0, idxb1)
    sd = (sd0, sd1)
    si = (si0, si1)
    so = (so0, so1)

    half = jnp.full((16,), 0.5, dtype=jnp.float32)

    def in_copies(g, b):
        r = row0 + g * R
        return (
            pltpu.make_async_copy(
                x_hbm.at[pl.ds(r * D, R * D)], data[b], sd[b]),
            pltpu.make_async_copy(
                idx_hbm.at[pl.ds(r * K, R * K)], idxb[b], si[b]),
        )

    def out_copy(g, b):
        r = row0 + g * R
        return pltpu.make_async_copy(
            data[b], out_hbm.at[pl.ds(r * D, R * D)], so[b])

    def start_in(g, b):
        a, c = in_copies(g, b)
        a.start()
        c.start()

    def wait_in(g, b):
        a, c = in_copies(g, b)
        a.wait()
        c.wait()

    def scatter(b):
        for r in range(R):
            for j in range(VPR):
                cols = idxb[b][pl.ds(r * K + j * 16, 16)]
                plsc.store_scatter(data[b], [cols + r * D], half)

    start_in(0, 0)
    start_in(1, 1)
    wait_in(0, 0)
    scatter(0)
    out_copy(0, 0).start()
    out_copy(0, 0).wait()
    start_in(2, 0)
    wait_in(1, 1)
    scatter(1)
    out_copy(1, 1).start()

    def loop_body(i, carry):
        g0 = i * 2
        out_copy(g0 - 1, 1).wait()
        start_in(g0 + 1, 1)
        wait_in(g0, 0)
        scatter(0)
        out_copy(g0, 0).start()
        out_copy(g0, 0).wait()
        start_in(g0 + 2, 0)
        wait_in(g0 + 1, 1)
        scatter(1)
        out_copy(g0 + 1, 1).start()
        return carry

    lax.fori_loop(1, CHUNKS // 2 - 1, loop_body, None)

    g = CHUNKS - 2
    out_copy(g - 1, 1).wait()
    start_in(g + 1, 1)
    wait_in(g, 0)
    scatter(0)
    out_copy(g, 0).start()
    out_copy(g, 0).wait()
    wait_in(g + 1, 1)
    scatter(1)
    out_copy(g + 1, 1).start()
    out_copy(g + 1, 1).wait()


_mesh = plsc.VectorSubcoreMesh(
    core_axis_name="c", subcore_axis_name="s", num_cores=NC, num_subcores=NS)

_scatter_call = pl.kernel(
    _body,
    out_type=jax.ShapeDtypeStruct((B * D,), jnp.float32),
    mesh=_mesh,
    compiler_params=pltpu.CompilerParams(needs_layout_passes=False),
    scratch_types=[
        pltpu.VMEM((R * D,), jnp.float32),
        pltpu.VMEM((R * D,), jnp.float32),
        pltpu.VMEM((R * K,), jnp.int32),
        pltpu.VMEM((R * K,), jnp.int32),
        pltpu.SemaphoreType.DMA,
        pltpu.SemaphoreType.DMA,
        pltpu.SemaphoreType.DMA,
        pltpu.SemaphoreType.DMA,
        pltpu.SemaphoreType.DMA,
        pltpu.SemaphoreType.DMA,
    ],
)


def kernel(x, index):
    flat = _scatter_call(
        x.reshape(B * D), index.astype(jnp.int32).reshape(B * K))
    return flat.reshape(B, D)


# speedup vs baseline: 14.7796x; 1.0023x over previous
import jax
import jax.numpy as jnp
from jax import lax
from jax.experimental import pallas as pl
from jax.experimental.pallas import tpu as pltpu
from jax.experimental.pallas import tpu_sc as plsc

B = 16384
D = 4096
K = 128
NC, NS = 2, 16
NW = NC * NS
ROWS_PER_W = B // NW
R = 8
CHUNKS = ROWS_PER_W // R
VPR = K // 16


def _body(x_hbm, idx_hbm, out_hbm,
          data0, data1, data2, idxb0, idxb1, idxb2,
          sd0, sd1, sd2, si0, si1, si2, so0, so1, so2):
    wid = lax.axis_index("s") * NC + lax.axis_index("c")
    row0 = wid * ROWS_PER_W

    data = (data0, data1, data2)
    idxb = (idxb0, idxb1, idxb2)
    sd = (sd0, sd1, sd2)
    si = (si0, si1, si2)
    so = (so0, so1, so2)

    half = jnp.full((16,), 0.5, dtype=jnp.float32)

    def in_copies(g, b):
        r = row0 + g * R
        return (
            pltpu.make_async_copy(
                x_hbm.at[pl.ds(r * D, R * D)], data[b], sd[b]),
            pltpu.make_async_copy(
                idx_hbm.at[pl.ds(r * K, R * K)], idxb[b], si[b]),
        )

    def out_copy(g, b):
        r = row0 + g * R
        return pltpu.make_async_copy(
            data[b], out_hbm.at[pl.ds(r * D, R * D)], so[b])

    def start_in(g, b):
        a, c = in_copies(g, b)
        a.start()
        c.start()

    def wait_in(g, b):
        a, c = in_copies(g, b)
        a.wait()
        c.wait()

    def scatter(b):
        for r in range(R):
            for j in range(VPR):
                cols = idxb[b][pl.ds(r * K + j * 16, 16)]
                plsc.store_scatter(data[b], [cols + r * D], half)

    start_in(0, 0)
    start_in(1, 1)
    start_in(2, 2)
    wait_in(0, 0)
    scatter(0)
    out_copy(0, 0).start()
    wait_in(1, 1)
    scatter(1)
    out_copy(1, 1).start()
    out_copy(0, 0).wait()
    start_in(3, 0)
    wait_in(2, 2)
    scatter(2)
    out_copy(2, 2).start()

    def loop_body(i, carry):
        g0 = i * 3
        for k in range(3):
            g = g0 + k
            b = k
            nb = (k + 1) % 3
            out_copy(g - 2, nb).wait()
            start_in(g + 1, nb)
            wait_in(g, b)
            scatter(b)
            out_copy(g, b).start()
        return carry

    lax.fori_loop(1, (CHUNKS - 1) // 3, loop_body, None)

    g = CHUNKS - 1
    out_copy(g - 2, 1).wait()
    wait_in(g, 0)
    scatter(0)
    out_copy(g, 0).start()
    out_copy(g - 1, 2).wait()
    out_copy(g, 0).wait()


_mesh = plsc.VectorSubcoreMesh(
    core_axis_name="c", subcore_axis_name="s", num_cores=NC, num_subcores=NS)

_scatter_call = pl.kernel(
    _body,
    out_type=jax.ShapeDtypeStruct((B * D,), jnp.float32),
    mesh=_mesh,
    compiler_params=pltpu.CompilerParams(needs_layout_passes=False),
    scratch_types=(
        [pltpu.VMEM((R * D,), jnp.float32)] * 3
        + [pltpu.VMEM((R * K,), jnp.int32)] * 3
        + [pltpu.SemaphoreType.DMA] * 9
    ),
)


def kernel(x, index):
    flat = _scatter_call(
        x.reshape(B * D), index.astype(jnp.int32).reshape(B * K))
    return flat.reshape(B, D)
